# baseline (speedup 1.0000x reference)
import jax
import jax.numpy as jnp
from jax import lax
from jax.experimental import pallas as pl
from jax.experimental.pallas import tpu as pltpu
from jax.experimental.pallas import tpu_sc as plsc

N_NODES = 10000
D_FEAT = 128
HIDDEN = 64
N_CLASSES = 2
N_EDGES = 320000

DW = 80
CHUNK = 128
NC = 2
NS = 16
NW = NC * NS
NCH = N_EDGES // CHUNK
BASE = NCH // NW
EXTRA = NCH % NW
ROWS_PER_TILE = N_NODES // NS
BLK = 2000
GRID = N_NODES // BLK


def _proj_body(x_ref, w_ref, e_ref, o_ref):
    o_ref[...] = (
        jnp.dot(x_ref[...], w_ref[...], preferred_element_type=jnp.float32)
        + e_ref[...]
    )


def _finish_body(a_ref, sel_ref, be_ref, wh_ref, bh_ref, o_ref):
    s = a_ref[0] + a_ref[1]
    deg = jnp.sum(s * sel_ref[...], axis=1, keepdims=True)
    inv = 1.0 / jnp.maximum(deg, 1.0)
    h = jnp.maximum(s * inv + be_ref[...], 0.0)
    o_ref[...] = (
        jnp.dot(h, wh_ref[...], preferred_element_type=jnp.float32)
        + bh_ref[...]
    )


def _edge_agg(y_hbm, ei_hbm, zeros_hbm, out_hbm,
              src_v, dst_v, rows_v, agg_sh, gsem0, gsem1, csem0, csem1):
    c = lax.axis_index("c")
    s = lax.axis_index("s")
    wid = s * NC + c
    gsems = (gsem0, gsem1)
    csems = (csem0, csem1)
    lo = wid * BASE + jnp.minimum(wid, EXTRA)
    nch = BASE + jnp.where(wid < EXTRA, 1, 0)

    def _gather(ch, b, issue):
        d = pltpu.make_async_copy(
            y_hbm.at[src_v.at[ch]], rows_v.at[b], gsems[b])
        d.start() if issue else d.wait()

    def _scatter(ch, b):
        d = pltpu.make_async_copy(
            rows_v.at[b], agg_sh.at[dst_v.at[ch]], csems[b])
        d.start(add=True)
        d.wait()

    with jax.named_scope("sc_init"):
        pltpu.sync_copy(ei_hbm.at[pl.ds(lo, BASE)], src_v.at[pl.ds(0, BASE)])
        pltpu.sync_copy(ei_hbm.at[pl.ds(NCH + lo, BASE)],
                        dst_v.at[pl.ds(0, BASE)])

        @pl.when(wid < EXTRA)
        def _():
            pltpu.sync_copy(ei_hbm.at[pl.ds(lo + BASE, 1)],
                            src_v.at[pl.ds(BASE, 1)])
            pltpu.sync_copy(ei_hbm.at[pl.ds(NCH + lo + BASE, 1)],
                            dst_v.at[pl.ds(BASE, 1)])

        _gather(0, 0, True)
        _gather(1, 1, True)
        pltpu.sync_copy(zeros_hbm.at[pl.ds(s * ROWS_PER_TILE, ROWS_PER_TILE)],
                        agg_sh.at[pl.ds(s * ROWS_PER_TILE, ROWS_PER_TILE)])
        plsc.subcore_barrier()

    with jax.named_scope("edge_loop"):

        @pl.loop(0, BASE // 2)
        def _pairs(j):
            for b in range(2):
                ch = 2 * j + b
                _gather(ch, b, False)
                _scatter(ch, b)

                @pl.when(ch + 2 < nch)
                def _():
                    _gather(ch + 2, b, True)

        @pl.when(wid < EXTRA)
        def _():
            _gather(BASE, 0, False)
            _scatter(BASE, 0)

    plsc.subcore_barrier()
    with jax.named_scope("publish"):
        pltpu.sync_copy(
            agg_sh.at[pl.ds(s * ROWS_PER_TILE, ROWS_PER_TILE)],
            out_hbm.at[c].at[pl.ds(s * ROWS_PER_TILE, ROWS_PER_TILE)])


def kernel(x, edge_index, W_enc, b_enc, W_head, b_head):
    f32 = jnp.float32
    w_pad = jnp.pad(W_enc, ((0, 0), (0, DW - HIDDEN)))
    ones_row = jnp.zeros((1, DW), f32).at[0, HIDDEN].set(1.0)
    ei = edge_index.reshape(2 * NCH, CHUNK)

    y = pl.pallas_call(
        _proj_body,
        grid=(GRID,),
        in_specs=[
            pl.BlockSpec((BLK, D_FEAT), lambda i: (i, 0)),
            pl.BlockSpec((D_FEAT, DW), lambda i: (0, 0)),
            pl.BlockSpec((1, DW), lambda i: (0, 0)),
        ],
        out_specs=pl.BlockSpec((BLK, DW), lambda i: (i, 0)),
        out_shape=jax.ShapeDtypeStruct((N_NODES, DW), f32),
    )(x, w_pad, ones_row)

    zeros = jnp.zeros((N_NODES, DW), f32)
    agg2 = pl.kernel(
        _edge_agg,
        out_type=jax.ShapeDtypeStruct((NC, N_NODES, DW), f32),
        mesh=plsc.VectorSubcoreMesh(core_axis_name="c", subcore_axis_name="s"),
        compiler_params=pltpu.CompilerParams(
            use_tc_tiling_on_sc=False, disable_bounds_checks=True),
        scratch_types=[
            pltpu.VMEM((BASE + 1, CHUNK), jnp.int32),
            pltpu.VMEM((BASE + 1, CHUNK), jnp.int32),
            pltpu.VMEM((2, CHUNK, DW), f32),
            pltpu.VMEM_SHARED((N_NODES, DW), f32),
            pltpu.SemaphoreType.DMA,
            pltpu.SemaphoreType.DMA,
            pltpu.SemaphoreType.DMA,
            pltpu.SemaphoreType.DMA,
        ],
    )(y, ei, zeros)

    sel = jnp.zeros((1, DW), f32).at[0, HIDDEN].set(1.0)
    be_pad = jnp.pad(b_enc, (0, DW - HIDDEN)).reshape(1, DW)
    wh_pad = jnp.pad(W_head, ((0, DW - HIDDEN), (0, 0)))
    bh_pad = b_head.reshape(1, N_CLASSES)
    logits = pl.pallas_call(
        _finish_body,
        grid=(GRID,),
        in_specs=[
            pl.BlockSpec((NC, BLK, DW), lambda i: (0, i, 0)),
            pl.BlockSpec((1, DW), lambda i: (0, 0)),
            pl.BlockSpec((1, DW), lambda i: (0, 0)),
            pl.BlockSpec((DW, N_CLASSES), lambda i: (0, 0)),
            pl.BlockSpec((1, N_CLASSES), lambda i: (0, 0)),
        ],
        out_specs=pl.BlockSpec((BLK, N_CLASSES), lambda i: (i, 0)),
        out_shape=jax.ShapeDtypeStruct((N_NODES, N_CLASSES), f32),
    )(agg2, sel, be_pad, wh_pad, bh_pad)

    return logits
